# baseline (speedup 1.0000x reference)
import functools

import jax
import jax.numpy as jnp
from jax import lax
from jax.experimental import pallas as pl
from jax.experimental.pallas import tpu as pltpu
from jax.experimental.pallas import tpu_sc as plsc

N = 10000
K = 32
DA = 128
DB = 16
E = N * K
_HALVES = 2
_NH = N // _HALVES
_EH = E // _HALVES

_NC = 2
_NS = 16
_NW = _NC * _NS
_EPW = _EH // _NW
_CHUNK = 200
_NCHUNK = _EPW // _CHUNK

_BA = 200
_RB = _BA * K
_GRID = _NH // _BA
_BN3 = 1000


def _sc_gather(nbr_flat, table):
  mesh = plsc.VectorSubcoreMesh(core_axis_name="c", subcore_axis_name="s")

  @functools.partial(
      pl.kernel,
      out_type=jax.ShapeDtypeStruct((_EH, DA), jnp.float32),
      mesh=mesh,
      compiler_params=pltpu.CompilerParams(use_tc_tiling_on_sc=True),
      scratch_types=[
          pltpu.VMEM((_EPW,), jnp.int32),
          pltpu.VMEM((2, _CHUNK, DA), jnp.float32),
          pltpu.SemaphoreType.DMA,
          pltpu.SemaphoreType.DMA,
          pltpu.SemaphoreType.DMA,
          pltpu.SemaphoreType.DMA,
      ],
  )
  def gather_kernel(idx_hbm, table_hbm, out_hbm, idx_v, rows_v, gs0, gs1, ws0, ws1):
    wid = lax.axis_index("s") * _NC + lax.axis_index("c")
    base = wid * _EPW
    pltpu.sync_copy(idx_hbm.at[pl.ds(base, _EPW)], idx_v)
    gsems = (gs0, gs1)
    wsems = (ws0, ws1)

    def issue_gather(j):
      bsel = j % 2
      return pltpu.async_copy(
          table_hbm.at[idx_v.at[pl.ds(j * _CHUNK, _CHUNK)]],
          rows_v.at[bsel], gsems[bsel])

    gathers = [issue_gather(0), None]
    writes = [None, None]
    for j in range(_NCHUNK):
      bsel = j % 2
      nb = (j + 1) % 2
      if j + 1 < _NCHUNK:
        if writes[nb] is not None:
          writes[nb].wait()
        gathers[nb] = issue_gather(j + 1)
      gathers[bsel].wait()
      writes[bsel] = pltpu.async_copy(
          rows_v.at[bsel], out_hbm.at[pl.ds(base + j * _CHUNK, _CHUNK)],
          wsems[bsel])
    writes[0].wait()
    writes[1].wait()

  return gather_kernel(nbr_flat, table)


def _edge_t(g_ref, bond_ref, atom_ref, ws_ref, wn_ref, wb_ref, b_ref):
  s = jnp.dot(atom_ref[...], ws_ref[...], preferred_element_type=jnp.float32)
  s = s + b_ref[...]
  t = jnp.dot(g_ref[...], wn_ref[...], preferred_element_type=jnp.float32)
  t = t + jnp.dot(bond_ref[...], wb_ref[...], preferred_element_type=jnp.float32)
  return t.reshape(_BA, K, 2 * DA) + s[:, None, :]


def _pass1_body(g_ref, bond_ref, atom_ref, ws_ref, wn_ref, wb_ref, b_ref,
                sum_ref, sumsq_ref):
  t3 = _edge_t(g_ref, bond_ref, atom_ref, ws_ref, wn_ref, wb_ref, b_ref)

  @pl.when(pl.program_id(0) == 0)
  def _init():
    sum_ref[...] = jnp.zeros_like(sum_ref)
    sumsq_ref[...] = jnp.zeros_like(sumsq_ref)

  sum_ref[...] += jnp.sum(t3, axis=(0, 1))[None, :]
  sumsq_ref[...] += jnp.sum(t3 * t3, axis=(0, 1))[None, :]


def _pass2_body(g_ref, bond_ref, atom_ref, ws_ref, wn_ref, wb_ref, b_ref,
                ns_ref, sum2_ref, sumsq2_ref):
  u = _edge_t(g_ref, bond_ref, atom_ref, ws_ref, wn_ref, wb_ref, b_ref)
  filt = jax.nn.sigmoid(u[:, :, :DA])
  core = jax.nn.softplus(u[:, :, DA:])
  ns = jnp.sum(filt * core, axis=1)
  ns_ref[...] = ns

  @pl.when(pl.program_id(0) == 0)
  def _init():
    sum2_ref[...] = jnp.zeros_like(sum2_ref)
    sumsq2_ref[...] = jnp.zeros_like(sumsq2_ref)

  sum2_ref[...] += jnp.sum(ns, axis=0)[None, :]
  sumsq2_ref[...] += jnp.sum(ns * ns, axis=0)[None, :]


def _pass3_body(atom_ref, ns_ref, a2_ref, c2_ref, out_ref):
  out_ref[...] = jax.nn.softplus(
      atom_ref[...] + ns_ref[...] * a2_ref[...] + c2_ref[...])


_full = lambda shp: pl.BlockSpec(shp, lambda i: (0, 0))

_EDGE_IN = [
    pl.BlockSpec((_RB, DA), lambda i: (i, 0)),
    pl.BlockSpec((_RB, DB), lambda i: (i, 0)),
    pl.BlockSpec((_BA, DA), lambda i: (i, 0)),
    _full((DA, 2 * DA)),
    _full((DA, 2 * DA)),
    _full((DB, 2 * DA)),
    _full((1, 2 * DA)),
]


def kernel(neighbor_indices, atom_features, bond_features, W, b,
           bn1_scale, bn1_offset, bn2_scale, bn2_offset):
  nbr_flat = neighbor_indices.astype(jnp.int32).reshape(E)
  bond_flat = bond_features.reshape(E, DB)
  w_self = W[:DA]
  w_nbr = W[DA:2 * DA]
  w_bond = W[2 * DA:]
  b2 = b.reshape(1, 2 * DA)

  nbr_h = [lax.slice_in_dim(nbr_flat, h * _EH, (h + 1) * _EH) for h in range(_HALVES)]
  bond_h = [lax.slice_in_dim(bond_flat, h * _EH, (h + 1) * _EH) for h in range(_HALVES)]
  atom_h = [lax.slice_in_dim(atom_features, h * _NH, (h + 1) * _NH) for h in range(_HALVES)]

  g_h = [_sc_gather(nbr_h[h], atom_features) for h in range(_HALVES)]

  pass1 = pl.pallas_call(
      _pass1_body,
      grid=(_GRID,),
      in_specs=_EDGE_IN,
      out_specs=[_full((1, 2 * DA)), _full((1, 2 * DA))],
      out_shape=[jax.ShapeDtypeStruct((1, 2 * DA), jnp.float32)] * 2,
  )
  sums = [pass1(g_h[h], bond_h[h], atom_h[h], w_self, w_nbr, w_bond, b2)
          for h in range(_HALVES)]
  colsum = sums[0][0] + sums[1][0]
  colsumsq = sums[0][1] + sums[1][1]

  mean1 = colsum / E
  var1 = colsumsq / E - mean1 * mean1
  a1 = bn1_scale.reshape(1, 2 * DA) * lax.rsqrt(var1 + 1e-5)
  c1 = bn1_offset.reshape(1, 2 * DA) - mean1 * a1

  ws_2 = w_self * a1
  wn_2 = w_nbr * a1
  wb_2 = w_bond * a1
  bias_2 = b2 * a1 + c1

  pass2 = pl.pallas_call(
      _pass2_body,
      grid=(_GRID,),
      in_specs=_EDGE_IN,
      out_specs=[
          pl.BlockSpec((_BA, DA), lambda i: (i, 0)),
          _full((1, DA)),
          _full((1, DA)),
      ],
      out_shape=[
          jax.ShapeDtypeStruct((_NH, DA), jnp.float32),
          jax.ShapeDtypeStruct((1, DA), jnp.float32),
          jax.ShapeDtypeStruct((1, DA), jnp.float32),
      ],
  )
  outs2 = [pass2(g_h[h], bond_h[h], atom_h[h], ws_2, wn_2, wb_2, bias_2)
           for h in range(_HALVES)]
  ns = jnp.concatenate([outs2[0][0], outs2[1][0]], axis=0)
  colsum2 = outs2[0][1] + outs2[1][1]
  colsumsq2 = outs2[0][2] + outs2[1][2]

  mean2 = colsum2 / N
  var2 = colsumsq2 / N - mean2 * mean2
  a2 = bn2_scale.reshape(1, DA) * lax.rsqrt(var2 + 1e-5)
  c2 = bn2_offset.reshape(1, DA) - mean2 * a2

  out = pl.pallas_call(
      _pass3_body,
      grid=(N // _BN3,),
      in_specs=[
          pl.BlockSpec((_BN3, DA), lambda i: (i, 0)),
          pl.BlockSpec((_BN3, DA), lambda i: (i, 0)),
          _full((1, DA)),
          _full((1, DA)),
      ],
      out_specs=pl.BlockSpec((_BN3, DA), lambda i: (i, 0)),
      out_shape=jax.ShapeDtypeStruct((N, DA), jnp.float32),
  )(atom_features, ns, a2, c2)

  return out


# speedup vs baseline: 1.0779x; 1.0779x over previous
import functools

import jax
import jax.numpy as jnp
from jax import lax
from jax.experimental import pallas as pl
from jax.experimental.pallas import tpu as pltpu
from jax.experimental.pallas import tpu_sc as plsc

N = 10000
K = 32
DA = 128
DB = 16
E = N * K
_HALVES = 2
_NH = N // _HALVES
_EH = E // _HALVES

_NC = 2
_NS = 16
_NW = _NC * _NS
_EPW = _EH // _NW
_CHUNK = 200
_NCHUNK = _EPW // _CHUNK

_BA = 200
_RB = _BA * K
_GRID = _NH // _BA
_BN3 = 1000


def _sc_gather(nbr_flat, table):
  mesh = plsc.VectorSubcoreMesh(core_axis_name="c", subcore_axis_name="s")

  @functools.partial(
      pl.kernel,
      out_type=jax.ShapeDtypeStruct((_EH, DA), jnp.float32),
      mesh=mesh,
      compiler_params=pltpu.CompilerParams(use_tc_tiling_on_sc=True),
      scratch_types=[
          pltpu.VMEM((_EPW,), jnp.int32),
          pltpu.VMEM((2, _CHUNK, DA), jnp.float32),
          pltpu.SemaphoreType.DMA,
          pltpu.SemaphoreType.DMA,
          pltpu.SemaphoreType.DMA,
          pltpu.SemaphoreType.DMA,
      ],
  )
  def gather_kernel(idx_hbm, table_hbm, out_hbm, idx_v, rows_v, gs0, gs1, ws0, ws1):
    wid = lax.axis_index("s") * _NC + lax.axis_index("c")
    base = wid * _EPW
    pltpu.sync_copy(idx_hbm.at[pl.ds(base, _EPW)], idx_v)
    gsems = (gs0, gs1)
    wsems = (ws0, ws1)

    def issue_gather(j):
      bsel = j % 2
      return pltpu.async_copy(
          table_hbm.at[idx_v.at[pl.ds(j * _CHUNK, _CHUNK)]],
          rows_v.at[bsel], gsems[bsel])

    gathers = [issue_gather(0), None]
    writes = [None, None]
    for j in range(_NCHUNK):
      bsel = j % 2
      nb = (j + 1) % 2
      if j + 1 < _NCHUNK:
        if writes[nb] is not None:
          writes[nb].wait()
        gathers[nb] = issue_gather(j + 1)
      gathers[bsel].wait()
      writes[bsel] = pltpu.async_copy(
          rows_v.at[bsel], out_hbm.at[pl.ds(base + j * _CHUNK, _CHUNK)],
          wsems[bsel])
    writes[0].wait()
    writes[1].wait()

  return gather_kernel(nbr_flat, table)


def _edge_t(g_ref, bond_ref, atom_ref, ws_ref, wn_ref, wb_ref, b_ref):
  s = jnp.dot(atom_ref[...], ws_ref[...], preferred_element_type=jnp.float32)
  s = s + b_ref[...]
  t = jnp.dot(g_ref[...], wn_ref[...], preferred_element_type=jnp.float32)
  t = t + jnp.dot(bond_ref[...], wb_ref[...], preferred_element_type=jnp.float32)
  return t.reshape(_BA, K, 2 * DA) + s[:, None, :]


def _pass1_body(g_ref, bond_ref, atom_ref, ws_ref, wn_ref, wb_ref, b_ref,
                sum_ref, sumsq_ref):
  t3 = _edge_t(g_ref, bond_ref, atom_ref, ws_ref, wn_ref, wb_ref, b_ref)

  @pl.when(pl.program_id(0) == 0)
  def _init():
    sum_ref[...] = jnp.zeros_like(sum_ref)
    sumsq_ref[...] = jnp.zeros_like(sumsq_ref)

  sum_ref[...] += jnp.sum(t3, axis=(0, 1))[None, :]
  sumsq_ref[...] += jnp.sum(t3 * t3, axis=(0, 1))[None, :]


def _pass2_body(g_ref, bond_ref, atom_ref, ws_ref, wn_ref, wb_ref, b_ref,
                ns_ref, sum2_ref, sumsq2_ref):
  u = _edge_t(g_ref, bond_ref, atom_ref, ws_ref, wn_ref, wb_ref, b_ref)
  filt = 1.0 / (1.0 + jnp.exp(-u[:, :, :DA]))
  core = jnp.log(1.0 + jnp.exp(u[:, :, DA:]))
  ns = jnp.sum(filt * core, axis=1)
  ns_ref[...] = ns

  @pl.when(pl.program_id(0) == 0)
  def _init():
    sum2_ref[...] = jnp.zeros_like(sum2_ref)
    sumsq2_ref[...] = jnp.zeros_like(sumsq2_ref)

  sum2_ref[...] += jnp.sum(ns, axis=0)[None, :]
  sumsq2_ref[...] += jnp.sum(ns * ns, axis=0)[None, :]


def _pass3_body(atom_ref, ns_ref, a2_ref, c2_ref, out_ref):
  out_ref[...] = jax.nn.softplus(
      atom_ref[...] + ns_ref[...] * a2_ref[...] + c2_ref[...])


_full = lambda shp: pl.BlockSpec(shp, lambda i: (0, 0))

_EDGE_IN = [
    pl.BlockSpec((_RB, DA), lambda i: (i, 0)),
    pl.BlockSpec((_RB, DB), lambda i: (i, 0)),
    pl.BlockSpec((_BA, DA), lambda i: (i, 0)),
    _full((DA, 2 * DA)),
    _full((DA, 2 * DA)),
    _full((DB, 2 * DA)),
    _full((1, 2 * DA)),
]


def kernel(neighbor_indices, atom_features, bond_features, W, b,
           bn1_scale, bn1_offset, bn2_scale, bn2_offset):
  nbr_flat = neighbor_indices.astype(jnp.int32).reshape(E)
  bond_flat = bond_features.reshape(E, DB)
  w_self = W[:DA]
  w_nbr = W[DA:2 * DA]
  w_bond = W[2 * DA:]
  b2 = b.reshape(1, 2 * DA)

  nbr_h = [lax.slice_in_dim(nbr_flat, h * _EH, (h + 1) * _EH) for h in range(_HALVES)]
  bond_h = [lax.slice_in_dim(bond_flat, h * _EH, (h + 1) * _EH) for h in range(_HALVES)]
  atom_h = [lax.slice_in_dim(atom_features, h * _NH, (h + 1) * _NH) for h in range(_HALVES)]

  g_h = [_sc_gather(nbr_h[h], atom_features) for h in range(_HALVES)]

  pass1 = pl.pallas_call(
      _pass1_body,
      grid=(_GRID,),
      in_specs=_EDGE_IN,
      out_specs=[_full((1, 2 * DA)), _full((1, 2 * DA))],
      out_shape=[jax.ShapeDtypeStruct((1, 2 * DA), jnp.float32)] * 2,
  )
  sums = [pass1(g_h[h], bond_h[h], atom_h[h], w_self, w_nbr, w_bond, b2)
          for h in range(_HALVES)]
  colsum = sums[0][0] + sums[1][0]
  colsumsq = sums[0][1] + sums[1][1]

  mean1 = colsum / E
  var1 = colsumsq / E - mean1 * mean1
  a1 = bn1_scale.reshape(1, 2 * DA) * lax.rsqrt(var1 + 1e-5)
  c1 = bn1_offset.reshape(1, 2 * DA) - mean1 * a1

  ws_2 = w_self * a1
  wn_2 = w_nbr * a1
  wb_2 = w_bond * a1
  bias_2 = b2 * a1 + c1

  pass2 = pl.pallas_call(
      _pass2_body,
      grid=(_GRID,),
      in_specs=_EDGE_IN,
      out_specs=[
          pl.BlockSpec((_BA, DA), lambda i: (i, 0)),
          _full((1, DA)),
          _full((1, DA)),
      ],
      out_shape=[
          jax.ShapeDtypeStruct((_NH, DA), jnp.float32),
          jax.ShapeDtypeStruct((1, DA), jnp.float32),
          jax.ShapeDtypeStruct((1, DA), jnp.float32),
      ],
  )
  outs2 = [pass2(g_h[h], bond_h[h], atom_h[h], ws_2, wn_2, wb_2, bias_2)
           for h in range(_HALVES)]
  ns = jnp.concatenate([outs2[0][0], outs2[1][0]], axis=0)
  colsum2 = outs2[0][1] + outs2[1][1]
  colsumsq2 = outs2[0][2] + outs2[1][2]

  mean2 = colsum2 / N
  var2 = colsumsq2 / N - mean2 * mean2
  a2 = bn2_scale.reshape(1, DA) * lax.rsqrt(var2 + 1e-5)
  c2 = bn2_offset.reshape(1, DA) - mean2 * a2

  out = pl.pallas_call(
      _pass3_body,
      grid=(N // _BN3,),
      in_specs=[
          pl.BlockSpec((_BN3, DA), lambda i: (i, 0)),
          pl.BlockSpec((_BN3, DA), lambda i: (i, 0)),
          _full((1, DA)),
          _full((1, DA)),
      ],
      out_specs=pl.BlockSpec((_BN3, DA), lambda i: (i, 0)),
      out_shape=jax.ShapeDtypeStruct((N, DA), jnp.float32),
  )(atom_features, ns, a2, c2)

  return out
